# Initial kernel scaffold; baseline (speedup 1.0000x reference)
import jax
import jax.numpy as jnp
from jax.experimental import pallas as pl


def kernel(x, edge_index, node_graph_ids, W_map, W1_conv, W1_res, W2_conv, W2_res):
    raise NotImplementedError("write your pallas kernel here")


# speedup vs baseline: 3.7382x; 3.7382x over previous
import functools

import jax
import jax.numpy as jnp
from jax import lax
from jax.experimental import pallas as pl
from jax.experimental.pallas import tpu as pltpu
from jax.experimental.pallas import tpu_sc as plsc

N = 50000
E = 800000
G = 1024
D_IN = 34
D_HID = 128
Q = 32
NQ = 4
NC = 2
NS = 16
K = 400
EPW = E // NS
NCHUNK = EPW // K
N_PAD = 50048
NROWS_W = N_PAD // NS
GROWS_W = G // NS
PK = 1000
NPCHUNK = N // PK
RB = 2000

f32 = jnp.float32

_mesh = plsc.VectorSubcoreMesh(core_axis_name="c", subcore_axis_name="s")
_sc_params = pltpu.CompilerParams(use_tc_tiling_on_sc=False)


def _edge_segsum_sc(m_qs, src, dst, zeros):
    @functools.partial(
        pl.kernel,
        out_type=[jax.ShapeDtypeStruct((N_PAD, Q), f32)] * NQ,
        mesh=_mesh,
        compiler_params=_sc_params,
        scratch_types=[
            pltpu.VMEM((K,), jnp.int32),
            pltpu.VMEM((K,), jnp.int32),
            pltpu.VMEM((K, Q), f32),
            pltpu.VMEM_SHARED((N_PAD, Q), f32),
            pltpu.SemaphoreType.DMA,
        ],
    )
    def k(m0, m1, m2, m3, src_h, dst_h, z_h, o0, o1, o2, o3,
          src_v, dst_v, rows_v, acc_sh, sem):
        cid = lax.axis_index("c")
        sid = lax.axis_index("s")
        m_refs = (m0, m1, m2, m3)
        o_refs = (o0, o1, o2, o3)

        def one_quarter(m_ref, o_ref):
            row0 = sid * NROWS_W
            pltpu.sync_copy(z_h.at[pl.ds(row0, NROWS_W)],
                            acc_sh.at[pl.ds(row0, NROWS_W)])
            plsc.subcore_barrier()

            @pl.loop(0, NCHUNK)
            def _(j):
                base = sid * EPW + j * K
                pltpu.sync_copy(src_h.at[pl.ds(base, K)], src_v)
                pltpu.sync_copy(dst_h.at[pl.ds(base, K)], dst_v)
                pltpu.async_copy(m_ref.at[src_v], rows_v, sem).wait()
                pltpu.sync_copy(rows_v, acc_sh.at[dst_v], add=True)

            plsc.subcore_barrier()
            pltpu.sync_copy(acc_sh.at[pl.ds(row0, NROWS_W)],
                            o_ref.at[pl.ds(row0, NROWS_W)])
            plsc.subcore_barrier()

        for c in range(NC):
            @pl.when(cid == c)
            def _():
                one_quarter(m_refs[2 * c], o_refs[2 * c])
                one_quarter(m_refs[2 * c + 1], o_refs[2 * c + 1])

    return k(*m_qs, src, dst, zeros)


def _pool_sc(h_qs, gids, zeros):
    @functools.partial(
        pl.kernel,
        out_type=[jax.ShapeDtypeStruct((G, Q), f32)] * NQ,
        mesh=_mesh,
        compiler_params=_sc_params,
        scratch_types=[
            pltpu.VMEM((PK,), jnp.int32),
            pltpu.VMEM((PK, Q), f32),
            pltpu.VMEM_SHARED((G, Q), f32),
            pltpu.SemaphoreType.DMA,
        ],
    )
    def k(h0, h1, h2, h3, g_h, z_h, o0, o1, o2, o3, ids_v, rows_v, acc_sh, sem):
        cid = lax.axis_index("c")
        sid = lax.axis_index("s")
        h_refs = (h0, h1, h2, h3)
        o_refs = (o0, o1, o2, o3)
        g0 = sid * GROWS_W
        nchunks = (NPCHUNK - sid + NS - 1) // NS

        def one_quarter(h_ref, o_ref):
            pltpu.sync_copy(z_h.at[pl.ds(g0, GROWS_W)],
                            acc_sh.at[pl.ds(g0, GROWS_W)])
            plsc.subcore_barrier()

            @pl.loop(0, nchunks)
            def _(j):
                base = (sid + j * NS) * PK
                pltpu.sync_copy(g_h.at[pl.ds(base, PK)], ids_v)
                pltpu.sync_copy(h_ref.at[pl.ds(base, PK)], rows_v)
                pltpu.sync_copy(rows_v, acc_sh.at[ids_v], add=True)

            plsc.subcore_barrier()
            pltpu.sync_copy(acc_sh.at[pl.ds(g0, GROWS_W)],
                            o_ref.at[pl.ds(g0, GROWS_W)])
            plsc.subcore_barrier()

        for c in range(NC):
            @pl.when(cid == c)
            def _():
                one_quarter(h_refs[2 * c], o_refs[2 * c])
                one_quarter(h_refs[2 * c + 1], o_refs[2 * c + 1])

    return k(*h_qs, gids, zeros)


def _tc_layer1(x, W_map, W1_conv, W1_res):
    def body(x_ref, wm_ref, wc_ref, wr_ref, m0, m1, m2, m3, r_ref):
        h0 = jnp.dot(x_ref[...], wm_ref[...], preferred_element_type=f32,
                     precision=lax.Precision.HIGHEST)
        m = jnp.dot(h0, wc_ref[...], preferred_element_type=f32,
                    precision=lax.Precision.HIGHEST)
        r_ref[...] = jax.nn.relu(jnp.dot(h0, wr_ref[...],
                                         preferred_element_type=f32,
                                         precision=lax.Precision.HIGHEST))
        m0[...] = m[:, 0 * Q:1 * Q]
        m1[...] = m[:, 1 * Q:2 * Q]
        m2[...] = m[:, 2 * Q:3 * Q]
        m3[...] = m[:, 3 * Q:4 * Q]

    return pl.pallas_call(
        body,
        grid=(N // RB,),
        in_specs=[
            pl.BlockSpec((RB, D_IN), lambda i: (i, 0)),
            pl.BlockSpec((D_IN, D_HID), lambda i: (0, 0)),
            pl.BlockSpec((D_HID, D_HID), lambda i: (0, 0)),
            pl.BlockSpec((D_HID, D_HID), lambda i: (0, 0)),
        ],
        out_specs=[pl.BlockSpec((RB, Q), lambda i: (i, 0))] * NQ
        + [pl.BlockSpec((RB, D_HID), lambda i: (i, 0))],
        out_shape=[jax.ShapeDtypeStruct((N, Q), f32)] * NQ
        + [jax.ShapeDtypeStruct((N, D_HID), f32)],
    )(x, W_map, W1_conv, W1_res)


def _tc_layer2(agg_qs, r1, W2_conv, W2_res):
    def body(a0, a1, a2, a3, r1_ref, wc_ref, wr_ref, m0, m1, m2, m3, r_ref):
        h = jnp.concatenate(
            [jax.nn.relu(a0[...]), jax.nn.relu(a1[...]),
             jax.nn.relu(a2[...]), jax.nn.relu(a3[...])], axis=1) + r1_ref[...]
        m = jnp.dot(h, wc_ref[...], preferred_element_type=f32,
                    precision=lax.Precision.HIGHEST)
        r_ref[...] = jax.nn.relu(jnp.dot(h, wr_ref[...],
                                         preferred_element_type=f32,
                                         precision=lax.Precision.HIGHEST))
        m0[...] = m[:, 0 * Q:1 * Q]
        m1[...] = m[:, 1 * Q:2 * Q]
        m2[...] = m[:, 2 * Q:3 * Q]
        m3[...] = m[:, 3 * Q:4 * Q]

    return pl.pallas_call(
        body,
        grid=(N // RB,),
        in_specs=[pl.BlockSpec((RB, Q), lambda i: (i, 0))] * NQ
        + [
            pl.BlockSpec((RB, D_HID), lambda i: (i, 0)),
            pl.BlockSpec((D_HID, D_HID), lambda i: (0, 0)),
            pl.BlockSpec((D_HID, D_HID), lambda i: (0, 0)),
        ],
        out_specs=[pl.BlockSpec((RB, Q), lambda i: (i, 0))] * NQ
        + [pl.BlockSpec((RB, D_HID), lambda i: (i, 0))],
        out_shape=[jax.ShapeDtypeStruct((N, Q), f32)] * NQ
        + [jax.ShapeDtypeStruct((N, D_HID), f32)],
    )(*agg_qs, r1, W2_conv, W2_res)


def _tc_h2(agg_qs, r2):
    def body(a0, a1, a2, a3, r_ref, o0, o1, o2, o3):
        r = r_ref[...]
        o0[...] = jax.nn.relu(a0[...]) + r[:, 0 * Q:1 * Q]
        o1[...] = jax.nn.relu(a1[...]) + r[:, 1 * Q:2 * Q]
        o2[...] = jax.nn.relu(a2[...]) + r[:, 2 * Q:3 * Q]
        o3[...] = jax.nn.relu(a3[...]) + r[:, 3 * Q:4 * Q]

    return pl.pallas_call(
        body,
        grid=(N // RB,),
        in_specs=[pl.BlockSpec((RB, Q), lambda i: (i, 0))] * NQ
        + [pl.BlockSpec((RB, D_HID), lambda i: (i, 0))],
        out_specs=[pl.BlockSpec((RB, Q), lambda i: (i, 0))] * NQ,
        out_shape=[jax.ShapeDtypeStruct((N, Q), f32)] * NQ,
    )(*agg_qs, r2)


def kernel(x, edge_index, node_graph_ids, W_map, W1_conv, W1_res, W2_conv,
           W2_res):
    src = edge_index[0]
    dst = edge_index[1]
    zeros = jnp.zeros((N_PAD, Q), f32)

    *m1_qs, r1 = _tc_layer1(x, W_map, W1_conv, W1_res)
    agg1_qs = _edge_segsum_sc(tuple(m1_qs), src, dst, zeros)
    *m2_qs, r2 = _tc_layer2(tuple(agg1_qs), r1, W2_conv, W2_res)
    agg2_qs = _edge_segsum_sc(tuple(m2_qs), src, dst, zeros)
    h2_qs = _tc_h2(tuple(agg2_qs), r2)
    out_qs = _pool_sc(tuple(h2_qs), node_graph_ids, zeros)
    return jnp.concatenate(out_qs, axis=1)


# speedup vs baseline: 5.2844x; 1.4136x over previous
import functools

import jax
import jax.numpy as jnp
from jax import lax
from jax.experimental import pallas as pl
from jax.experimental.pallas import tpu as pltpu
from jax.experimental.pallas import tpu_sc as plsc

N = 50000
E = 800000
G = 1024
D_IN = 34
D_HID = 128
Q = 32
NQ = 4
NC = 2
NS = 16
K = 400
SK = 2000
SUBS = SK // K
EPW = E // NS
NSUP = EPW // SK
N_PAD = 50048
NROWS_W = N_PAD // NS
GROWS_W = G // NS
PK = 1000
NPCHUNK = N // PK
RB = 2000

f32 = jnp.float32

_mesh = plsc.VectorSubcoreMesh(core_axis_name="c", subcore_axis_name="s")
_sc_params = pltpu.CompilerParams(use_tc_tiling_on_sc=False)


def _edge_segsum_sc(m_qs, src, dst, zeros):
    @functools.partial(
        pl.kernel,
        out_type=[jax.ShapeDtypeStruct((N_PAD, Q), f32)] * NQ,
        mesh=_mesh,
        compiler_params=_sc_params,
        scratch_types=[
            pltpu.VMEM((SK,), jnp.int32),
            pltpu.VMEM((SK,), jnp.int32),
            pltpu.VMEM((K, Q), f32),
            pltpu.VMEM((K, Q), f32),
            pltpu.VMEM_SHARED((N_PAD, Q), f32),
            pltpu.SemaphoreType.DMA,
            pltpu.SemaphoreType.DMA,
        ],
    )
    def k(m0, m1, m2, m3, src_h, dst_h, z_h, o0, o1, o2, o3,
          src_v, dst_v, rows0_v, rows1_v, acc_sh, sem0, sem1):
        cid = lax.axis_index("c")
        sid = lax.axis_index("s")
        m_refs = (m0, m1, m2, m3)
        o_refs = (o0, o1, o2, o3)

        def one_quarter(m_ref, o_ref):
            row0 = sid * NROWS_W
            pltpu.sync_copy(z_h.at[pl.ds(row0, NROWS_W)],
                            acc_sh.at[pl.ds(row0, NROWS_W)])
            plsc.subcore_barrier()

            rows = (rows0_v, rows1_v)
            sems = (sem0, sem1)

            @pl.loop(0, NSUP)
            def _(sup):
                base = sid * EPW + sup * SK
                pltpu.sync_copy(src_h.at[pl.ds(base, SK)], src_v)
                pltpu.sync_copy(dst_h.at[pl.ds(base, SK)], dst_v)
                handles = [pltpu.async_copy(
                    m_ref.at[src_v.at[pl.ds(0, K)]], rows[0], sems[0])]
                for t in range(SUBS):
                    if t + 1 < SUBS:
                        handles.append(pltpu.async_copy(
                            m_ref.at[src_v.at[pl.ds((t + 1) * K, K)]],
                            rows[(t + 1) % 2], sems[(t + 1) % 2]))
                    handles[t].wait()
                    pltpu.sync_copy(rows[t % 2],
                                    acc_sh.at[dst_v.at[pl.ds(t * K, K)]],
                                    add=True)

            plsc.subcore_barrier()
            pltpu.sync_copy(acc_sh.at[pl.ds(row0, NROWS_W)],
                            o_ref.at[pl.ds(row0, NROWS_W)])
            plsc.subcore_barrier()

        for c in range(NC):
            @pl.when(cid == c)
            def _():
                one_quarter(m_refs[2 * c], o_refs[2 * c])
                one_quarter(m_refs[2 * c + 1], o_refs[2 * c + 1])

    return k(*m_qs, src, dst, zeros)


def _pool_sc(h_qs, gids, zeros):
    @functools.partial(
        pl.kernel,
        out_type=[jax.ShapeDtypeStruct((G, Q), f32)] * NQ,
        mesh=_mesh,
        compiler_params=_sc_params,
        scratch_types=[
            pltpu.VMEM((PK,), jnp.int32),
            pltpu.VMEM((PK, Q), f32),
            pltpu.VMEM_SHARED((G, Q), f32),
            pltpu.SemaphoreType.DMA,
        ],
    )
    def k(h0, h1, h2, h3, g_h, z_h, o0, o1, o2, o3, ids_v, rows_v, acc_sh, sem):
        cid = lax.axis_index("c")
        sid = lax.axis_index("s")
        h_refs = (h0, h1, h2, h3)
        o_refs = (o0, o1, o2, o3)
        g0 = sid * GROWS_W
        nchunks = (NPCHUNK - sid + NS - 1) // NS

        def one_quarter(h_ref, o_ref):
            pltpu.sync_copy(z_h.at[pl.ds(g0, GROWS_W)],
                            acc_sh.at[pl.ds(g0, GROWS_W)])
            plsc.subcore_barrier()

            @pl.loop(0, nchunks)
            def _(j):
                base = (sid + j * NS) * PK
                pltpu.sync_copy(g_h.at[pl.ds(base, PK)], ids_v)
                pltpu.sync_copy(h_ref.at[pl.ds(base, PK)], rows_v)
                pltpu.sync_copy(rows_v, acc_sh.at[ids_v], add=True)

            plsc.subcore_barrier()
            pltpu.sync_copy(acc_sh.at[pl.ds(g0, GROWS_W)],
                            o_ref.at[pl.ds(g0, GROWS_W)])
            plsc.subcore_barrier()

        for c in range(NC):
            @pl.when(cid == c)
            def _():
                one_quarter(h_refs[2 * c], o_refs[2 * c])
                one_quarter(h_refs[2 * c + 1], o_refs[2 * c + 1])

    return k(*h_qs, gids, zeros)


def _tc_layer1(x, W_map, W1_conv, W1_res):
    def body(x_ref, wm_ref, wc_ref, wr_ref, m0, m1, m2, m3, r_ref):
        h0 = jnp.dot(x_ref[...], wm_ref[...], preferred_element_type=f32,
                     precision=lax.Precision.HIGHEST)
        m = jnp.dot(h0, wc_ref[...], preferred_element_type=f32,
                    precision=lax.Precision.HIGHEST)
        r_ref[...] = jax.nn.relu(jnp.dot(h0, wr_ref[...],
                                         preferred_element_type=f32,
                                         precision=lax.Precision.HIGHEST))
        m0[...] = m[:, 0 * Q:1 * Q]
        m1[...] = m[:, 1 * Q:2 * Q]
        m2[...] = m[:, 2 * Q:3 * Q]
        m3[...] = m[:, 3 * Q:4 * Q]

    return pl.pallas_call(
        body,
        grid=(N // RB,),
        in_specs=[
            pl.BlockSpec((RB, D_IN), lambda i: (i, 0)),
            pl.BlockSpec((D_IN, D_HID), lambda i: (0, 0)),
            pl.BlockSpec((D_HID, D_HID), lambda i: (0, 0)),
            pl.BlockSpec((D_HID, D_HID), lambda i: (0, 0)),
        ],
        out_specs=[pl.BlockSpec((RB, Q), lambda i: (i, 0))] * NQ
        + [pl.BlockSpec((RB, D_HID), lambda i: (i, 0))],
        out_shape=[jax.ShapeDtypeStruct((N, Q), f32)] * NQ
        + [jax.ShapeDtypeStruct((N, D_HID), f32)],
    )(x, W_map, W1_conv, W1_res)


def _tc_layer2(agg_qs, r1, W2_conv, W2_res):
    def body(a0, a1, a2, a3, r1_ref, wc_ref, wr_ref, m0, m1, m2, m3, r_ref):
        h = jnp.concatenate(
            [jax.nn.relu(a0[...]), jax.nn.relu(a1[...]),
             jax.nn.relu(a2[...]), jax.nn.relu(a3[...])], axis=1) + r1_ref[...]
        m = jnp.dot(h, wc_ref[...], preferred_element_type=f32,
                    precision=lax.Precision.HIGHEST)
        r_ref[...] = jax.nn.relu(jnp.dot(h, wr_ref[...],
                                         preferred_element_type=f32,
                                         precision=lax.Precision.HIGHEST))
        m0[...] = m[:, 0 * Q:1 * Q]
        m1[...] = m[:, 1 * Q:2 * Q]
        m2[...] = m[:, 2 * Q:3 * Q]
        m3[...] = m[:, 3 * Q:4 * Q]

    return pl.pallas_call(
        body,
        grid=(N // RB,),
        in_specs=[pl.BlockSpec((RB, Q), lambda i: (i, 0))] * NQ
        + [
            pl.BlockSpec((RB, D_HID), lambda i: (i, 0)),
            pl.BlockSpec((D_HID, D_HID), lambda i: (0, 0)),
            pl.BlockSpec((D_HID, D_HID), lambda i: (0, 0)),
        ],
        out_specs=[pl.BlockSpec((RB, Q), lambda i: (i, 0))] * NQ
        + [pl.BlockSpec((RB, D_HID), lambda i: (i, 0))],
        out_shape=[jax.ShapeDtypeStruct((N, Q), f32)] * NQ
        + [jax.ShapeDtypeStruct((N, D_HID), f32)],
    )(*agg_qs, r1, W2_conv, W2_res)


def _tc_h2(agg_qs, r2):
    def body(a0, a1, a2, a3, r_ref, o0, o1, o2, o3):
        r = r_ref[...]
        o0[...] = jax.nn.relu(a0[...]) + r[:, 0 * Q:1 * Q]
        o1[...] = jax.nn.relu(a1[...]) + r[:, 1 * Q:2 * Q]
        o2[...] = jax.nn.relu(a2[...]) + r[:, 2 * Q:3 * Q]
        o3[...] = jax.nn.relu(a3[...]) + r[:, 3 * Q:4 * Q]

    return pl.pallas_call(
        body,
        grid=(N // RB,),
        in_specs=[pl.BlockSpec((RB, Q), lambda i: (i, 0))] * NQ
        + [pl.BlockSpec((RB, D_HID), lambda i: (i, 0))],
        out_specs=[pl.BlockSpec((RB, Q), lambda i: (i, 0))] * NQ,
        out_shape=[jax.ShapeDtypeStruct((N, Q), f32)] * NQ,
    )(*agg_qs, r2)


def kernel(x, edge_index, node_graph_ids, W_map, W1_conv, W1_res, W2_conv,
           W2_res):
    src = edge_index[0]
    dst = edge_index[1]
    zeros = jnp.zeros((N_PAD, Q), f32)

    *m1_qs, r1 = _tc_layer1(x, W_map, W1_conv, W1_res)
    agg1_qs = _edge_segsum_sc(tuple(m1_qs), src, dst, zeros)
    *m2_qs, r2 = _tc_layer2(tuple(agg1_qs), r1, W2_conv, W2_res)
    agg2_qs = _edge_segsum_sc(tuple(m2_qs), src, dst, zeros)
    h2_qs = _tc_h2(tuple(agg2_qs), r2)
    out_qs = _pool_sc(tuple(h2_qs), node_graph_ids, zeros)
    return jnp.concatenate(out_qs, axis=1)


# speedup vs baseline: 5.5020x; 1.0412x over previous
import functools

import jax
import jax.numpy as jnp
from jax import lax
from jax.experimental import pallas as pl
from jax.experimental.pallas import tpu as pltpu
from jax.experimental.pallas import tpu_sc as plsc

N = 50000
E = 800000
G = 1024
D_IN = 34
D_HID = 128
Q = 32
NQ = 4
NC = 2
NS = 16
K = 400
SK = 2000
SUBS = SK // K
EPW = E // NS
NSUP = EPW // SK
N_PAD = 50048
NROWS_W = N_PAD // NS
GROWS_W = G // NS
PK = 1000
NPCHUNK = N // PK
RB = 2000

f32 = jnp.float32

_mesh = plsc.VectorSubcoreMesh(core_axis_name="c", subcore_axis_name="s")
_sc_params = pltpu.CompilerParams(use_tc_tiling_on_sc=False)


def _edge_segsum_sc(m_qs, src, dst, zeros):
    @functools.partial(
        pl.kernel,
        out_type=[jax.ShapeDtypeStruct((N_PAD, Q), f32)] * NQ,
        mesh=_mesh,
        compiler_params=_sc_params,
        scratch_types=[
            pltpu.VMEM((SK,), jnp.int32),
            pltpu.VMEM((SK,), jnp.int32),
            pltpu.VMEM((K, Q), f32),
            pltpu.VMEM((K, Q), f32),
            pltpu.VMEM_SHARED((N_PAD, Q), f32),
            pltpu.SemaphoreType.DMA,
            pltpu.SemaphoreType.DMA,
            pltpu.SemaphoreType.DMA,
            pltpu.SemaphoreType.DMA,
            pltpu.SemaphoreType.DMA,
            pltpu.SemaphoreType.DMA,
        ],
    )
    def k(m0, m1, m2, m3, src_h, dst_h, z_h, o0, o1, o2, o3,
          src_v, dst_v, rows0_v, rows1_v, acc_sh,
          gsem0, gsem1, ssem0, ssem1, ixsem0, ixsem1):
        cid = lax.axis_index("c")
        sid = lax.axis_index("s")
        m_refs = (m0, m1, m2, m3)
        o_refs = (o0, o1, o2, o3)

        def one_quarter(m_ref, o_ref):
            row0 = sid * NROWS_W
            pltpu.sync_copy(z_h.at[pl.ds(row0, NROWS_W)],
                            acc_sh.at[pl.ds(row0, NROWS_W)])
            plsc.subcore_barrier()

            rows = (rows0_v, rows1_v)
            gsems = (gsem0, gsem1)
            ssems = (ssem0, ssem1)

            @pl.loop(0, NSUP)
            def _(sup):
                base = sid * EPW + sup * SK
                hs = pltpu.async_copy(src_h.at[pl.ds(base, SK)], src_v, ixsem0)
                hd = pltpu.async_copy(dst_h.at[pl.ds(base, SK)], dst_v, ixsem1)
                hs.wait()
                hd.wait()
                g = [pltpu.async_copy(
                    m_ref.at[src_v.at[pl.ds(0, K)]], rows[0], gsems[0])]
                s = [None] * SUBS
                for t in range(SUBS):
                    b = t % 2
                    if t + 1 < SUBS:
                        if t >= 1:
                            s[t - 1].wait()
                        g.append(pltpu.async_copy(
                            m_ref.at[src_v.at[pl.ds((t + 1) * K, K)]],
                            rows[(t + 1) % 2], gsems[(t + 1) % 2]))
                    g[t].wait()
                    s[t] = pltpu.async_copy(
                        rows[b], acc_sh.at[dst_v.at[pl.ds(t * K, K)]],
                        ssems[b], add=True)
                s[SUBS - 2].wait()
                s[SUBS - 1].wait()

            plsc.subcore_barrier()
            pltpu.sync_copy(acc_sh.at[pl.ds(row0, NROWS_W)],
                            o_ref.at[pl.ds(row0, NROWS_W)])
            plsc.subcore_barrier()

        for c in range(NC):
            @pl.when(cid == c)
            def _():
                one_quarter(m_refs[2 * c], o_refs[2 * c])
                one_quarter(m_refs[2 * c + 1], o_refs[2 * c + 1])

    return k(*m_qs, src, dst, zeros)


def _pool_sc(h_qs, gids, zeros):
    @functools.partial(
        pl.kernel,
        out_type=[jax.ShapeDtypeStruct((G, Q), f32)] * NQ,
        mesh=_mesh,
        compiler_params=_sc_params,
        scratch_types=[
            pltpu.VMEM((PK,), jnp.int32),
            pltpu.VMEM((PK, Q), f32),
            pltpu.VMEM_SHARED((G, Q), f32),
            pltpu.SemaphoreType.DMA,
        ],
    )
    def k(h0, h1, h2, h3, g_h, z_h, o0, o1, o2, o3, ids_v, rows_v, acc_sh, sem):
        cid = lax.axis_index("c")
        sid = lax.axis_index("s")
        h_refs = (h0, h1, h2, h3)
        o_refs = (o0, o1, o2, o3)
        g0 = sid * GROWS_W
        nchunks = (NPCHUNK - sid + NS - 1) // NS

        def one_quarter(h_ref, o_ref):
            pltpu.sync_copy(z_h.at[pl.ds(g0, GROWS_W)],
                            acc_sh.at[pl.ds(g0, GROWS_W)])
            plsc.subcore_barrier()

            @pl.loop(0, nchunks)
            def _(j):
                base = (sid + j * NS) * PK
                pltpu.sync_copy(g_h.at[pl.ds(base, PK)], ids_v)
                pltpu.sync_copy(h_ref.at[pl.ds(base, PK)], rows_v)
                pltpu.sync_copy(rows_v, acc_sh.at[ids_v], add=True)

            plsc.subcore_barrier()
            pltpu.sync_copy(acc_sh.at[pl.ds(g0, GROWS_W)],
                            o_ref.at[pl.ds(g0, GROWS_W)])
            plsc.subcore_barrier()

        for c in range(NC):
            @pl.when(cid == c)
            def _():
                one_quarter(h_refs[2 * c], o_refs[2 * c])
                one_quarter(h_refs[2 * c + 1], o_refs[2 * c + 1])

    return k(*h_qs, gids, zeros)


def _tc_layer1(x, W_map, W1_conv, W1_res):
    def body(x_ref, wm_ref, wc_ref, wr_ref, m0, m1, m2, m3, r_ref):
        h0 = jnp.dot(x_ref[...], wm_ref[...], preferred_element_type=f32,
                     precision=lax.Precision.HIGHEST)
        m = jnp.dot(h0, wc_ref[...], preferred_element_type=f32,
                    precision=lax.Precision.HIGHEST)
        r_ref[...] = jax.nn.relu(jnp.dot(h0, wr_ref[...],
                                         preferred_element_type=f32,
                                         precision=lax.Precision.HIGHEST))
        m0[...] = m[:, 0 * Q:1 * Q]
        m1[...] = m[:, 1 * Q:2 * Q]
        m2[...] = m[:, 2 * Q:3 * Q]
        m3[...] = m[:, 3 * Q:4 * Q]

    return pl.pallas_call(
        body,
        grid=(N // RB,),
        in_specs=[
            pl.BlockSpec((RB, D_IN), lambda i: (i, 0)),
            pl.BlockSpec((D_IN, D_HID), lambda i: (0, 0)),
            pl.BlockSpec((D_HID, D_HID), lambda i: (0, 0)),
            pl.BlockSpec((D_HID, D_HID), lambda i: (0, 0)),
        ],
        out_specs=[pl.BlockSpec((RB, Q), lambda i: (i, 0))] * NQ
        + [pl.BlockSpec((RB, D_HID), lambda i: (i, 0))],
        out_shape=[jax.ShapeDtypeStruct((N, Q), f32)] * NQ
        + [jax.ShapeDtypeStruct((N, D_HID), f32)],
    )(x, W_map, W1_conv, W1_res)


def _tc_layer2(agg_qs, r1, W2_conv, W2_res):
    def body(a0, a1, a2, a3, r1_ref, wc_ref, wr_ref, m0, m1, m2, m3, r_ref):
        h = jnp.concatenate(
            [jax.nn.relu(a0[...]), jax.nn.relu(a1[...]),
             jax.nn.relu(a2[...]), jax.nn.relu(a3[...])], axis=1) + r1_ref[...]
        m = jnp.dot(h, wc_ref[...], preferred_element_type=f32,
                    precision=lax.Precision.HIGHEST)
        r_ref[...] = jax.nn.relu(jnp.dot(h, wr_ref[...],
                                         preferred_element_type=f32,
                                         precision=lax.Precision.HIGHEST))
        m0[...] = m[:, 0 * Q:1 * Q]
        m1[...] = m[:, 1 * Q:2 * Q]
        m2[...] = m[:, 2 * Q:3 * Q]
        m3[...] = m[:, 3 * Q:4 * Q]

    return pl.pallas_call(
        body,
        grid=(N // RB,),
        in_specs=[pl.BlockSpec((RB, Q), lambda i: (i, 0))] * NQ
        + [
            pl.BlockSpec((RB, D_HID), lambda i: (i, 0)),
            pl.BlockSpec((D_HID, D_HID), lambda i: (0, 0)),
            pl.BlockSpec((D_HID, D_HID), lambda i: (0, 0)),
        ],
        out_specs=[pl.BlockSpec((RB, Q), lambda i: (i, 0))] * NQ
        + [pl.BlockSpec((RB, D_HID), lambda i: (i, 0))],
        out_shape=[jax.ShapeDtypeStruct((N, Q), f32)] * NQ
        + [jax.ShapeDtypeStruct((N, D_HID), f32)],
    )(*agg_qs, r1, W2_conv, W2_res)


def _tc_h2(agg_qs, r2):
    def body(a0, a1, a2, a3, r_ref, o0, o1, o2, o3):
        r = r_ref[...]
        o0[...] = jax.nn.relu(a0[...]) + r[:, 0 * Q:1 * Q]
        o1[...] = jax.nn.relu(a1[...]) + r[:, 1 * Q:2 * Q]
        o2[...] = jax.nn.relu(a2[...]) + r[:, 2 * Q:3 * Q]
        o3[...] = jax.nn.relu(a3[...]) + r[:, 3 * Q:4 * Q]

    return pl.pallas_call(
        body,
        grid=(N // RB,),
        in_specs=[pl.BlockSpec((RB, Q), lambda i: (i, 0))] * NQ
        + [pl.BlockSpec((RB, D_HID), lambda i: (i, 0))],
        out_specs=[pl.BlockSpec((RB, Q), lambda i: (i, 0))] * NQ,
        out_shape=[jax.ShapeDtypeStruct((N, Q), f32)] * NQ,
    )(*agg_qs, r2)


def kernel(x, edge_index, node_graph_ids, W_map, W1_conv, W1_res, W2_conv,
           W2_res):
    src = edge_index[0]
    dst = edge_index[1]
    zeros = jnp.zeros((N_PAD, Q), f32)

    *m1_qs, r1 = _tc_layer1(x, W_map, W1_conv, W1_res)
    agg1_qs = _edge_segsum_sc(tuple(m1_qs), src, dst, zeros)
    *m2_qs, r2 = _tc_layer2(tuple(agg1_qs), r1, W2_conv, W2_res)
    agg2_qs = _edge_segsum_sc(tuple(m2_qs), src, dst, zeros)
    h2_qs = _tc_h2(tuple(agg2_qs), r2)
    out_qs = _pool_sc(tuple(h2_qs), node_graph_ids, zeros)
    return jnp.concatenate(out_qs, axis=1)


# speedup vs baseline: 7.4567x; 1.3553x over previous
import functools

import jax
import jax.numpy as jnp
from jax import lax
from jax.experimental import pallas as pl
from jax.experimental.pallas import tpu as pltpu
from jax.experimental.pallas import tpu_sc as plsc

N = 50000
E = 800000
G = 1024
D_IN = 34
D_HID = 128
Q = 32
NQ = 4
NC = 2
NS = 16
K = 400
SK = 2000
SUBS = SK // K
EPW = E // NS
NSUP = EPW // SK
N_PAD = 50048
NROWS_W = N_PAD // NS
GROWS_W = G // NS
PK = 1000
NPCHUNK = N // PK
RB = 2000

f32 = jnp.float32

_mesh = plsc.VectorSubcoreMesh(core_axis_name="c", subcore_axis_name="s")
_sc_params = pltpu.CompilerParams(use_tc_tiling_on_sc=False)


def _edge_segsum_sc(m, src4, dst, zeros):
    @functools.partial(
        pl.kernel,
        out_type=jax.ShapeDtypeStruct((N_PAD, D_HID), f32),
        mesh=_mesh,
        compiler_params=_sc_params,
        scratch_types=[
            pltpu.VMEM((SK,), jnp.int32),
            pltpu.VMEM((SK,), jnp.int32),
            pltpu.VMEM((K, Q), f32),
            pltpu.VMEM((K, Q), f32),
            pltpu.VMEM_SHARED((N_PAD, Q), f32),
            pltpu.SemaphoreType.DMA,
            pltpu.SemaphoreType.DMA,
            pltpu.SemaphoreType.DMA,
            pltpu.SemaphoreType.DMA,
            pltpu.SemaphoreType.DMA,
            pltpu.SemaphoreType.DMA,
        ],
    )
    def k(m_h, s0_h, s1_h, s2_h, s3_h, dst_h, z_h, o_h,
          src_v, dst_v, rows0_v, rows1_v, acc_sh,
          gsem0, gsem1, ssem0, ssem1, ixsem0, ixsem1):
        cid = lax.axis_index("c")
        sid = lax.axis_index("s")
        src_refs = (s0_h, s1_h, s2_h, s3_h)

        def one_quarter(q):
            src_h = src_refs[q]
            col = q * Q
            row0 = sid * NROWS_W
            pltpu.sync_copy(z_h.at[pl.ds(row0, NROWS_W), pl.ds(0, Q)],
                            acc_sh.at[pl.ds(row0, NROWS_W)])
            plsc.subcore_barrier()

            rows = (rows0_v, rows1_v)
            gsems = (gsem0, gsem1)
            ssems = (ssem0, ssem1)

            @pl.loop(0, NSUP)
            def _(sup):
                base = sid * EPW + sup * SK
                hs = pltpu.async_copy(src_h.at[pl.ds(base, SK)], src_v, ixsem0)
                hd = pltpu.async_copy(dst_h.at[pl.ds(base, SK)], dst_v, ixsem1)
                hs.wait()
                hd.wait()
                g = [pltpu.async_copy(
                    m_h.at[src_v.at[pl.ds(0, K)]], rows[0], gsems[0])]
                s = [None] * SUBS
                for t in range(SUBS):
                    b = t % 2
                    if t + 1 < SUBS:
                        if t >= 1:
                            s[t - 1].wait()
                        g.append(pltpu.async_copy(
                            m_h.at[src_v.at[pl.ds((t + 1) * K, K)]],
                            rows[(t + 1) % 2], gsems[(t + 1) % 2]))
                    g[t].wait()
                    s[t] = pltpu.async_copy(
                        rows[b], acc_sh.at[dst_v.at[pl.ds(t * K, K)]],
                        ssems[b], add=True)
                s[SUBS - 2].wait()
                s[SUBS - 1].wait()

            plsc.subcore_barrier()
            pltpu.sync_copy(acc_sh.at[pl.ds(row0, NROWS_W)],
                            o_h.at[pl.ds(row0, NROWS_W), pl.ds(col, Q)])
            plsc.subcore_barrier()

        for c in range(NC):
            @pl.when(cid == c)
            def _():
                one_quarter(2 * c)
                one_quarter(2 * c + 1)

    return k(m.reshape(N * NQ, Q), *src4, dst, zeros)


def _pool_sc(h, gids, zeros):
    H = D_HID // NC

    @functools.partial(
        pl.kernel,
        out_type=jax.ShapeDtypeStruct((G, D_HID), f32),
        mesh=_mesh,
        compiler_params=_sc_params,
        scratch_types=[
            pltpu.VMEM((PK,), jnp.int32),
            pltpu.VMEM((PK, H), f32),
            pltpu.VMEM_SHARED((G, H), f32),
            pltpu.SemaphoreType.DMA,
        ],
    )
    def k(h_h, g_h, z_h, o_h, ids_v, rows_v, acc_sh, sem):
        cid = lax.axis_index("c")
        sid = lax.axis_index("s")
        g0 = sid * GROWS_W
        nchunks = (NPCHUNK - sid + NS - 1) // NS

        def one_half(c):
            col = c * H
            pltpu.sync_copy(z_h.at[pl.ds(g0, GROWS_W), pl.ds(0, H)],
                            acc_sh.at[pl.ds(g0, GROWS_W)])
            plsc.subcore_barrier()

            @pl.loop(0, nchunks)
            def _(j):
                base = (sid + j * NS) * PK
                pltpu.sync_copy(g_h.at[pl.ds(base, PK)], ids_v)
                pltpu.sync_copy(h_h.at[pl.ds(base, PK), pl.ds(col, H)], rows_v)
                pltpu.sync_copy(rows_v, acc_sh.at[ids_v], add=True)

            plsc.subcore_barrier()
            pltpu.sync_copy(acc_sh.at[pl.ds(g0, GROWS_W)],
                            o_h.at[pl.ds(g0, GROWS_W), pl.ds(col, H)])
            plsc.subcore_barrier()

        for c in range(NC):
            @pl.when(cid == c)
            def _():
                one_half(c)

    return k(h, gids, zeros)


def _tc_layer1(x, W_map, W1_conv, W1_res):
    def body(x_ref, wm_ref, wc_ref, wr_ref, m_ref, r_ref):
        h0 = jnp.dot(x_ref[...], wm_ref[...], preferred_element_type=f32,
                     precision=lax.Precision.HIGHEST)
        m_ref[...] = jnp.dot(h0, wc_ref[...], preferred_element_type=f32,
                             precision=lax.Precision.HIGHEST)
        r_ref[...] = jax.nn.relu(jnp.dot(h0, wr_ref[...],
                                         preferred_element_type=f32,
                                         precision=lax.Precision.HIGHEST))

    return pl.pallas_call(
        body,
        grid=(N // RB,),
        in_specs=[
            pl.BlockSpec((RB, D_IN), lambda i: (i, 0)),
            pl.BlockSpec((D_IN, D_HID), lambda i: (0, 0)),
            pl.BlockSpec((D_HID, D_HID), lambda i: (0, 0)),
            pl.BlockSpec((D_HID, D_HID), lambda i: (0, 0)),
        ],
        out_specs=[pl.BlockSpec((RB, D_HID), lambda i: (i, 0))] * 2,
        out_shape=[jax.ShapeDtypeStruct((N, D_HID), f32)] * 2,
    )(x, W_map, W1_conv, W1_res)


def _tc_layer2(agg1, r1, W2_conv, W2_res):
    def body(a_ref, r1_ref, wc_ref, wr_ref, m_ref, r_ref):
        h = jax.nn.relu(a_ref[...]) + r1_ref[...]
        m_ref[...] = jnp.dot(h, wc_ref[...], preferred_element_type=f32,
                             precision=lax.Precision.HIGHEST)
        r_ref[...] = jax.nn.relu(jnp.dot(h, wr_ref[...],
                                         preferred_element_type=f32,
                                         precision=lax.Precision.HIGHEST))

    return pl.pallas_call(
        body,
        grid=(N // RB,),
        in_specs=[
            pl.BlockSpec((RB, D_HID), lambda i: (i, 0)),
            pl.BlockSpec((RB, D_HID), lambda i: (i, 0)),
            pl.BlockSpec((D_HID, D_HID), lambda i: (0, 0)),
            pl.BlockSpec((D_HID, D_HID), lambda i: (0, 0)),
        ],
        out_specs=[pl.BlockSpec((RB, D_HID), lambda i: (i, 0))] * 2,
        out_shape=[jax.ShapeDtypeStruct((N, D_HID), f32)] * 2,
    )(agg1, r1, W2_conv, W2_res)


def _tc_h2(agg2, r2):
    def body(a_ref, r_ref, o_ref):
        o_ref[...] = jax.nn.relu(a_ref[...]) + r_ref[...]

    return pl.pallas_call(
        body,
        grid=(N // RB,),
        in_specs=[pl.BlockSpec((RB, D_HID), lambda i: (i, 0))] * 2,
        out_specs=pl.BlockSpec((RB, D_HID), lambda i: (i, 0)),
        out_shape=jax.ShapeDtypeStruct((N, D_HID), f32),
    )(agg2, r2)


def kernel(x, edge_index, node_graph_ids, W_map, W1_conv, W1_res, W2_conv,
           W2_res):
    src = edge_index[0]
    dst = edge_index[1]
    src4 = tuple(src * NQ + q for q in range(NQ))
    zeros = jnp.zeros((N_PAD, D_HID // NC), f32)

    m1, r1 = _tc_layer1(x, W_map, W1_conv, W1_res)
    agg1 = _edge_segsum_sc(m1, src4, dst, zeros)
    m2, r2 = _tc_layer2(agg1, r1, W2_conv, W2_res)
    agg2 = _edge_segsum_sc(m2, src4, dst, zeros)
    h2 = _tc_h2(agg2, r2)
    return _pool_sc(h2, node_graph_ids, zeros)


# speedup vs baseline: 7.4663x; 1.0013x over previous
import functools

import jax
import jax.numpy as jnp
from jax import lax
from jax.experimental import pallas as pl
from jax.experimental.pallas import tpu as pltpu
from jax.experimental.pallas import tpu_sc as plsc

N = 50000
E = 800000
G = 1024
D_IN = 34
D_HID = 128
Q = 32
NQ = 4
NC = 2
NS = 16
K = 200
SK = 1000
SUBS = SK // K
NBUF = 4
EPW = E // NS
NSUP = EPW // SK
N_PAD = 50048
NROWS_W = N_PAD // NS
GROWS_W = G // NS
PK = 1000
NPCHUNK = N // PK
RB = 2000

f32 = jnp.float32

_mesh = plsc.VectorSubcoreMesh(core_axis_name="c", subcore_axis_name="s")
_sc_params = pltpu.CompilerParams(use_tc_tiling_on_sc=False)


def _edge_segsum_sc(m, src4, dst, zeros):
    @functools.partial(
        pl.kernel,
        out_type=jax.ShapeDtypeStruct((N_PAD, D_HID), f32),
        mesh=_mesh,
        compiler_params=_sc_params,
        scratch_types=[
            pltpu.VMEM((SK,), jnp.int32),
            pltpu.VMEM((SK,), jnp.int32),
            pltpu.VMEM((SK,), jnp.int32),
            pltpu.VMEM((SK,), jnp.int32),
            pltpu.VMEM((K, Q), f32),
            pltpu.VMEM((K, Q), f32),
            pltpu.VMEM((K, Q), f32),
            pltpu.VMEM((K, Q), f32),
            pltpu.VMEM_SHARED((N_PAD, Q), f32),
        ] + [pltpu.SemaphoreType.DMA] * 10,
    )
    def k(m_h, s0_h, s1_h, s2_h, s3_h, dst_h, z_h, o_h,
          srcA_v, dstA_v, srcB_v, dstB_v,
          rows0_v, rows1_v, rows2_v, rows3_v, acc_sh,
          gsem0, gsem1, gsem2, gsem3,
          ssem0, ssem1, ssem2, ssem3, ixsemA, ixsemB):
        cid = lax.axis_index("c")
        sid = lax.axis_index("s")
        src_refs = (s0_h, s1_h, s2_h, s3_h)
        rows = (rows0_v, rows1_v, rows2_v, rows3_v)
        gsems = (gsem0, gsem1, gsem2, gsem3)
        ssems = (ssem0, ssem1, ssem2, ssem3)
        ixbufs = ((srcA_v, dstA_v, ixsemA), (srcB_v, dstB_v, ixsemB))

        def one_quarter(q):
            src_h = src_refs[q]
            col = q * Q
            row0 = sid * NROWS_W
            base0 = sid * EPW

            def issue_idx(p, base):
                sv, dv, sem = ixbufs[p]
                pltpu.async_copy(src_h.at[pl.ds(base, SK)], sv, sem)
                pltpu.async_copy(dst_h.at[pl.ds(base, SK)], dv, sem)

            def wait_idx(p):
                sv, dv, sem = ixbufs[p]
                pltpu.make_async_copy(src_h.at[pl.ds(base0, SK)], sv, sem).wait()
                pltpu.make_async_copy(dst_h.at[pl.ds(base0, SK)], dv, sem).wait()

            def process_super(p, base):
                sv, dv, _ = ixbufs[p]
                g = [pltpu.async_copy(
                    m_h.at[sv.at[pl.ds(t * K, K)]], rows[t], gsems[t])
                    for t in range(NBUF)]
                s = [None] * SUBS
                for t in range(SUBS):
                    b = t % NBUF
                    g[t].wait()
                    s[t] = pltpu.async_copy(
                        rows[b], acc_sh.at[dv.at[pl.ds(t * K, K)]],
                        ssems[b], add=True)
                    if t + NBUF < SUBS:
                        s[t].wait()
                        g.append(pltpu.async_copy(
                            m_h.at[sv.at[pl.ds((t + NBUF) * K, K)]],
                            rows[b], gsems[b]))
                for t in range(max(0, SUBS - NBUF), SUBS):
                    s[t].wait()

            pltpu.sync_copy(z_h.at[pl.ds(row0, NROWS_W), pl.ds(0, Q)],
                            acc_sh.at[pl.ds(row0, NROWS_W)])
            issue_idx(0, base0)
            plsc.subcore_barrier()

            @pl.loop(0, NSUP, step=2)
            def _(sup):
                issue_idx(1, base0 + (sup + 1) * SK)
                wait_idx(0)
                process_super(0, base0 + sup * SK)

                @pl.when(sup + 2 < NSUP)
                def _():
                    issue_idx(0, base0 + (sup + 2) * SK)

                wait_idx(1)
                process_super(1, base0 + (sup + 1) * SK)

            plsc.subcore_barrier()
            pltpu.sync_copy(acc_sh.at[pl.ds(row0, NROWS_W)],
                            o_h.at[pl.ds(row0, NROWS_W), pl.ds(col, Q)])
            plsc.subcore_barrier()

        for c in range(NC):
            @pl.when(cid == c)
            def _():
                one_quarter(2 * c)
                one_quarter(2 * c + 1)

    return k(m.reshape(N * NQ, Q), *src4, dst, zeros)


def _pool_sc(h, gids, zeros):
    H = D_HID // NC

    @functools.partial(
        pl.kernel,
        out_type=jax.ShapeDtypeStruct((G, D_HID), f32),
        mesh=_mesh,
        compiler_params=_sc_params,
        scratch_types=[
            pltpu.VMEM((PK,), jnp.int32),
            pltpu.VMEM((PK, H), f32),
            pltpu.VMEM_SHARED((G, H), f32),
            pltpu.SemaphoreType.DMA,
        ],
    )
    def k(h_h, g_h, z_h, o_h, ids_v, rows_v, acc_sh, sem):
        cid = lax.axis_index("c")
        sid = lax.axis_index("s")
        g0 = sid * GROWS_W
        nchunks = (NPCHUNK - sid + NS - 1) // NS

        def one_half(c):
            col = c * H
            pltpu.sync_copy(z_h.at[pl.ds(g0, GROWS_W), pl.ds(0, H)],
                            acc_sh.at[pl.ds(g0, GROWS_W)])
            plsc.subcore_barrier()

            @pl.loop(0, nchunks)
            def _(j):
                base = (sid + j * NS) * PK
                pltpu.sync_copy(g_h.at[pl.ds(base, PK)], ids_v)
                pltpu.sync_copy(h_h.at[pl.ds(base, PK), pl.ds(col, H)], rows_v)
                pltpu.sync_copy(rows_v, acc_sh.at[ids_v], add=True)

            plsc.subcore_barrier()
            pltpu.sync_copy(acc_sh.at[pl.ds(g0, GROWS_W)],
                            o_h.at[pl.ds(g0, GROWS_W), pl.ds(col, H)])
            plsc.subcore_barrier()

        for c in range(NC):
            @pl.when(cid == c)
            def _():
                one_half(c)

    return k(h, gids, zeros)


def _tc_layer1(x, W_map, W1_conv, W1_res):
    def body(x_ref, wm_ref, wc_ref, wr_ref, m_ref, r_ref):
        h0 = jnp.dot(x_ref[...], wm_ref[...], preferred_element_type=f32,
                     precision=lax.Precision.HIGHEST)
        m_ref[...] = jnp.dot(h0, wc_ref[...], preferred_element_type=f32,
                             precision=lax.Precision.HIGHEST)
        r_ref[...] = jax.nn.relu(jnp.dot(h0, wr_ref[...],
                                         preferred_element_type=f32,
                                         precision=lax.Precision.HIGHEST))

    return pl.pallas_call(
        body,
        grid=(N // RB,),
        in_specs=[
            pl.BlockSpec((RB, D_IN), lambda i: (i, 0)),
            pl.BlockSpec((D_IN, D_HID), lambda i: (0, 0)),
            pl.BlockSpec((D_HID, D_HID), lambda i: (0, 0)),
            pl.BlockSpec((D_HID, D_HID), lambda i: (0, 0)),
        ],
        out_specs=[pl.BlockSpec((RB, D_HID), lambda i: (i, 0))] * 2,
        out_shape=[jax.ShapeDtypeStruct((N, D_HID), f32)] * 2,
    )(x, W_map, W1_conv, W1_res)


def _tc_layer2(agg1, r1, W2_conv, W2_res):
    def body(a_ref, r1_ref, wc_ref, wr_ref, m_ref, r_ref):
        h = jax.nn.relu(a_ref[...]) + r1_ref[...]
        m_ref[...] = jnp.dot(h, wc_ref[...], preferred_element_type=f32,
                             precision=lax.Precision.HIGHEST)
        r_ref[...] = jax.nn.relu(jnp.dot(h, wr_ref[...],
                                         preferred_element_type=f32,
                                         precision=lax.Precision.HIGHEST))

    return pl.pallas_call(
        body,
        grid=(N // RB,),
        in_specs=[
            pl.BlockSpec((RB, D_HID), lambda i: (i, 0)),
            pl.BlockSpec((RB, D_HID), lambda i: (i, 0)),
            pl.BlockSpec((D_HID, D_HID), lambda i: (0, 0)),
            pl.BlockSpec((D_HID, D_HID), lambda i: (0, 0)),
        ],
        out_specs=[pl.BlockSpec((RB, D_HID), lambda i: (i, 0))] * 2,
        out_shape=[jax.ShapeDtypeStruct((N, D_HID), f32)] * 2,
    )(agg1, r1, W2_conv, W2_res)


def _tc_h2(agg2, r2):
    def body(a_ref, r_ref, o_ref):
        o_ref[...] = jax.nn.relu(a_ref[...]) + r_ref[...]

    return pl.pallas_call(
        body,
        grid=(N // RB,),
        in_specs=[pl.BlockSpec((RB, D_HID), lambda i: (i, 0))] * 2,
        out_specs=pl.BlockSpec((RB, D_HID), lambda i: (i, 0)),
        out_shape=jax.ShapeDtypeStruct((N, D_HID), f32),
    )(agg2, r2)


def kernel(x, edge_index, node_graph_ids, W_map, W1_conv, W1_res, W2_conv,
           W2_res):
    src = edge_index[0]
    dst = edge_index[1]
    src4 = tuple(src * NQ + q for q in range(NQ))
    zeros = jnp.zeros((N_PAD, D_HID // NC), f32)

    m1, r1 = _tc_layer1(x, W_map, W1_conv, W1_res)
    agg1 = _edge_segsum_sc(m1, src4, dst, zeros)
    m2, r2 = _tc_layer2(agg1, r1, W2_conv, W2_res)
    agg2 = _edge_segsum_sc(m2, src4, dst, zeros)
    h2 = _tc_h2(agg2, r2)
    return _pool_sc(h2, node_graph_ids, zeros)


# speedup vs baseline: 7.8332x; 1.0491x over previous
import functools

import jax
import jax.numpy as jnp
from jax import lax
from jax.experimental import pallas as pl
from jax.experimental.pallas import tpu as pltpu
from jax.experimental.pallas import tpu_sc as plsc

N = 50000
E = 800000
G = 1024
D_IN = 34
D_HID = 128
Q = 32
NQ = 4
NC = 2
NS = 16
K = 200
SK = 1000
SUBS = SK // K
NBUF = 4
EPW = E // NS
NSUP = EPW // SK
N_PAD = 50048
NROWS_W = N_PAD // NS
GROWS_W = G // NS
PK = 1000
NPCHUNK = N // PK
RB = 2000

f32 = jnp.float32

_mesh = plsc.VectorSubcoreMesh(core_axis_name="c", subcore_axis_name="s")
_sc_params = pltpu.CompilerParams(use_tc_tiling_on_sc=False)


def _edge_segsum_sc(m, src4, dst, zeros):
    @functools.partial(
        pl.kernel,
        out_type=jax.ShapeDtypeStruct((N_PAD, D_HID), f32),
        mesh=_mesh,
        compiler_params=_sc_params,
        scratch_types=[
            pltpu.VMEM((SK,), jnp.int32),
            pltpu.VMEM((SK,), jnp.int32),
            pltpu.VMEM((SK,), jnp.int32),
            pltpu.VMEM((SK,), jnp.int32),
            pltpu.VMEM((K, Q), f32),
            pltpu.VMEM((K, Q), f32),
            pltpu.VMEM((K, Q), f32),
            pltpu.VMEM((K, Q), f32),
            pltpu.VMEM_SHARED((N_PAD, Q), f32),
        ] + [pltpu.SemaphoreType.DMA] * 10,
    )
    def k(m_h, s0_h, s1_h, s2_h, s3_h, dst_h, z_h, o_h,
          srcA_v, dstA_v, srcB_v, dstB_v,
          rows0_v, rows1_v, rows2_v, rows3_v, acc_sh,
          gsem0, gsem1, gsem2, gsem3,
          ssem0, ssem1, ssem2, ssem3, ixsemA, ixsemB):
        cid = lax.axis_index("c")
        sid = lax.axis_index("s")
        src_refs = (s0_h, s1_h, s2_h, s3_h)
        rows = (rows0_v, rows1_v, rows2_v, rows3_v)
        gsems = (gsem0, gsem1, gsem2, gsem3)
        ssems = (ssem0, ssem1, ssem2, ssem3)
        ixbufs = ((srcA_v, dstA_v, ixsemA), (srcB_v, dstB_v, ixsemB))

        def one_quarter(q):
            src_h = src_refs[q]
            col = q * Q
            row0 = sid * NROWS_W
            base0 = sid * EPW

            def issue_idx(p, base):
                sv, dv, sem = ixbufs[p]
                pltpu.async_copy(src_h.at[pl.ds(base, SK)], sv, sem)
                pltpu.async_copy(dst_h.at[pl.ds(base, SK)], dv, sem)

            def wait_idx(p):
                sv, dv, sem = ixbufs[p]
                pltpu.make_async_copy(src_h.at[pl.ds(base0, SK)], sv, sem).wait()
                pltpu.make_async_copy(dst_h.at[pl.ds(base0, SK)], dv, sem).wait()

            def process_super(p, base):
                sv, dv, _ = ixbufs[p]
                g = [pltpu.async_copy(
                    m_h.at[sv.at[pl.ds(t * K, K)]], rows[t], gsems[t])
                    for t in range(NBUF)]
                s = [None] * SUBS
                for t in range(SUBS):
                    b = t % NBUF
                    g[t].wait()
                    s[t] = pltpu.async_copy(
                        rows[b], acc_sh.at[dv.at[pl.ds(t * K, K)]],
                        ssems[b], add=True)
                    if t + NBUF < SUBS:
                        s[t].wait()
                        g.append(pltpu.async_copy(
                            m_h.at[sv.at[pl.ds((t + NBUF) * K, K)]],
                            rows[b], gsems[b]))
                for t in range(max(0, SUBS - NBUF), SUBS):
                    s[t].wait()

            pltpu.sync_copy(z_h.at[pl.ds(row0, NROWS_W), pl.ds(0, Q)],
                            acc_sh.at[pl.ds(row0, NROWS_W)])
            issue_idx(0, base0)
            plsc.subcore_barrier()

            @pl.loop(0, NSUP, step=2)
            def _(sup):
                issue_idx(1, base0 + (sup + 1) * SK)
                wait_idx(0)
                process_super(0, base0 + sup * SK)

                @pl.when(sup + 2 < NSUP)
                def _():
                    issue_idx(0, base0 + (sup + 2) * SK)

                wait_idx(1)
                process_super(1, base0 + (sup + 1) * SK)

            plsc.subcore_barrier()
            pltpu.sync_copy(acc_sh.at[pl.ds(row0, NROWS_W)],
                            o_h.at[pl.ds(row0, NROWS_W), pl.ds(col, Q)])
            plsc.subcore_barrier()

        for c in range(NC):
            @pl.when(cid == c)
            def _():
                one_quarter(2 * c)
                one_quarter(2 * c + 1)

    return k(m.reshape(N * NQ, Q), *src4, dst, zeros)


def _pool_sc(h, gids, zeros):
    H = D_HID // NC

    @functools.partial(
        pl.kernel,
        out_type=jax.ShapeDtypeStruct((G, D_HID), f32),
        mesh=_mesh,
        compiler_params=_sc_params,
        scratch_types=[
            pltpu.VMEM((PK,), jnp.int32),
            pltpu.VMEM((PK, H), f32),
            pltpu.VMEM_SHARED((G, H), f32),
            pltpu.SemaphoreType.DMA,
        ],
    )
    def k(h_h, g_h, z_h, o_h, ids_v, rows_v, acc_sh, sem):
        cid = lax.axis_index("c")
        sid = lax.axis_index("s")
        g0 = sid * GROWS_W
        nchunks = (NPCHUNK - sid + NS - 1) // NS

        def one_half(c):
            col = c * H
            pltpu.sync_copy(z_h.at[pl.ds(g0, GROWS_W), pl.ds(0, H)],
                            acc_sh.at[pl.ds(g0, GROWS_W)])
            plsc.subcore_barrier()

            @pl.loop(0, nchunks)
            def _(j):
                base = (sid + j * NS) * PK
                pltpu.sync_copy(g_h.at[pl.ds(base, PK)], ids_v)
                pltpu.sync_copy(h_h.at[pl.ds(base, PK), pl.ds(col, H)], rows_v)
                pltpu.sync_copy(rows_v, acc_sh.at[ids_v], add=True)

            plsc.subcore_barrier()
            pltpu.sync_copy(acc_sh.at[pl.ds(g0, GROWS_W)],
                            o_h.at[pl.ds(g0, GROWS_W), pl.ds(col, H)])
            plsc.subcore_barrier()

        for c in range(NC):
            @pl.when(cid == c)
            def _():
                one_half(c)

    return k(h, gids, zeros)


def _tc_layer1(x, W_map, W1_conv, W1_res):
    def body(x_ref, wm_ref, wc_ref, wr_ref, m_ref, r_ref):
        h0 = jnp.dot(x_ref[...], wm_ref[...], preferred_element_type=f32)
        m_ref[...] = jnp.dot(h0, wc_ref[...], preferred_element_type=f32)
        r_ref[...] = jax.nn.relu(jnp.dot(h0, wr_ref[...],
                                         preferred_element_type=f32))

    return pl.pallas_call(
        body,
        grid=(N // RB,),
        in_specs=[
            pl.BlockSpec((RB, D_IN), lambda i: (i, 0)),
            pl.BlockSpec((D_IN, D_HID), lambda i: (0, 0)),
            pl.BlockSpec((D_HID, D_HID), lambda i: (0, 0)),
            pl.BlockSpec((D_HID, D_HID), lambda i: (0, 0)),
        ],
        out_specs=[pl.BlockSpec((RB, D_HID), lambda i: (i, 0))] * 2,
        out_shape=[jax.ShapeDtypeStruct((N, D_HID), f32)] * 2,
    )(x, W_map, W1_conv, W1_res)


def _tc_layer2(agg1, r1, W2_conv, W2_res):
    def body(a_ref, r1_ref, wc_ref, wr_ref, m_ref, r_ref):
        h = jax.nn.relu(a_ref[...]) + r1_ref[...]
        m_ref[...] = jnp.dot(h, wc_ref[...], preferred_element_type=f32)
        r_ref[...] = jax.nn.relu(jnp.dot(h, wr_ref[...],
                                         preferred_element_type=f32))

    return pl.pallas_call(
        body,
        grid=(N // RB,),
        in_specs=[
            pl.BlockSpec((RB, D_HID), lambda i: (i, 0)),
            pl.BlockSpec((RB, D_HID), lambda i: (i, 0)),
            pl.BlockSpec((D_HID, D_HID), lambda i: (0, 0)),
            pl.BlockSpec((D_HID, D_HID), lambda i: (0, 0)),
        ],
        out_specs=[pl.BlockSpec((RB, D_HID), lambda i: (i, 0))] * 2,
        out_shape=[jax.ShapeDtypeStruct((N, D_HID), f32)] * 2,
    )(agg1, r1, W2_conv, W2_res)


def _tc_h2(agg2, r2):
    def body(a_ref, r_ref, o_ref):
        o_ref[...] = jax.nn.relu(a_ref[...]) + r_ref[...]

    return pl.pallas_call(
        body,
        grid=(N // RB,),
        in_specs=[pl.BlockSpec((RB, D_HID), lambda i: (i, 0))] * 2,
        out_specs=pl.BlockSpec((RB, D_HID), lambda i: (i, 0)),
        out_shape=jax.ShapeDtypeStruct((N, D_HID), f32),
    )(agg2, r2)


def kernel(x, edge_index, node_graph_ids, W_map, W1_conv, W1_res, W2_conv,
           W2_res):
    src = edge_index[0]
    dst = edge_index[1]
    src4 = tuple(src * NQ + q for q in range(NQ))
    zeros = jnp.zeros((N_PAD, D_HID // NC), f32)

    m1, r1 = _tc_layer1(x, W_map, W1_conv, W1_res)
    agg1 = _edge_segsum_sc(m1, src4, dst, zeros)
    m2, r2 = _tc_layer2(agg1, r1, W2_conv, W2_res)
    agg2 = _edge_segsum_sc(m2, src4, dst, zeros)
    h2 = _tc_h2(agg2, r2)
    return _pool_sc(h2, node_graph_ids, zeros)


# speedup vs baseline: 8.0944x; 1.0333x over previous
import functools

import jax
import jax.numpy as jnp
from jax import lax
from jax.experimental import pallas as pl
from jax.experimental.pallas import tpu as pltpu
from jax.experimental.pallas import tpu_sc as plsc

N = 50000
E = 800000
G = 1024
D_IN = 34
D_HID = 128
Q = 32
NQ = 4
NC = 2
NS = 16
K = 200
SK = 1000
SUBS = SK // K
NBUF = 4
EPW = E // NS
NSUP = EPW // SK
N_PAD = 50048
NROWS_W = N_PAD // NS
GROWS_W = G // NS
PK = 1000
NPCHUNK = N // PK
RB = 2000

f32 = jnp.float32

_mesh = plsc.VectorSubcoreMesh(core_axis_name="c", subcore_axis_name="s")
_sc_params = pltpu.CompilerParams(use_tc_tiling_on_sc=False)


def _edge_segsum_sc(m, src4, dst, zeros):
    @functools.partial(
        pl.kernel,
        out_type=jax.ShapeDtypeStruct((N_PAD, D_HID), f32),
        mesh=_mesh,
        compiler_params=_sc_params,
        scratch_types=[
            pltpu.VMEM((SK,), jnp.int32),
            pltpu.VMEM((SK,), jnp.int32),
            pltpu.VMEM((SK,), jnp.int32),
            pltpu.VMEM((SK,), jnp.int32),
            pltpu.VMEM((K, Q), f32),
            pltpu.VMEM((K, Q), f32),
            pltpu.VMEM((K, Q), f32),
            pltpu.VMEM((K, Q), f32),
            pltpu.VMEM_SHARED((N_PAD, Q), f32),
        ] + [pltpu.SemaphoreType.DMA] * 10,
    )
    def k(m_h, s0_h, s1_h, s2_h, s3_h, dst_h, z_h, o_h,
          srcA_v, dstA_v, srcB_v, dstB_v,
          rows0_v, rows1_v, rows2_v, rows3_v, acc_sh,
          gsem0, gsem1, gsem2, gsem3,
          ssem0, ssem1, ssem2, ssem3, ixsemA, ixsemB):
        cid = lax.axis_index("c")
        sid = lax.axis_index("s")
        src_refs = (s0_h, s1_h, s2_h, s3_h)
        rows = (rows0_v, rows1_v, rows2_v, rows3_v)
        gsems = (gsem0, gsem1, gsem2, gsem3)
        ssems = (ssem0, ssem1, ssem2, ssem3)
        ixbufs = ((srcA_v, dstA_v, ixsemA), (srcB_v, dstB_v, ixsemB))

        def one_quarter(q):
            src_h = src_refs[q]
            col = q * Q
            row0 = sid * NROWS_W
            base0 = sid * EPW

            def issue_idx(p, base):
                sv, dv, sem = ixbufs[p]
                pltpu.async_copy(src_h.at[pl.ds(base, SK)], sv, sem)
                pltpu.async_copy(dst_h.at[pl.ds(base, SK)], dv, sem)

            def wait_idx(p):
                sv, dv, sem = ixbufs[p]
                pltpu.make_async_copy(src_h.at[pl.ds(base0, SK)], sv, sem).wait()
                pltpu.make_async_copy(dst_h.at[pl.ds(base0, SK)], dv, sem).wait()

            def process_super(p, base):
                sv, dv, _ = ixbufs[p]
                g = [pltpu.async_copy(
                    m_h.at[sv.at[pl.ds(t * K, K)]], rows[t], gsems[t])
                    for t in range(NBUF)]
                s = [None] * SUBS
                for t in range(SUBS):
                    b = t % NBUF
                    g[t].wait()
                    s[t] = pltpu.async_copy(
                        rows[b], acc_sh.at[dv.at[pl.ds(t * K, K)]],
                        ssems[b], add=True)
                    if t + NBUF < SUBS:
                        s[t].wait()
                        g.append(pltpu.async_copy(
                            m_h.at[sv.at[pl.ds((t + NBUF) * K, K)]],
                            rows[b], gsems[b]))
                for t in range(max(0, SUBS - NBUF), SUBS):
                    s[t].wait()

            pltpu.sync_copy(z_h.at[pl.ds(row0, NROWS_W), pl.ds(0, Q)],
                            acc_sh.at[pl.ds(row0, NROWS_W)])
            issue_idx(0, base0)
            plsc.subcore_barrier()

            @pl.loop(0, NSUP, step=2)
            def _(sup):
                issue_idx(1, base0 + (sup + 1) * SK)
                wait_idx(0)
                process_super(0, base0 + sup * SK)

                @pl.when(sup + 2 < NSUP)
                def _():
                    issue_idx(0, base0 + (sup + 2) * SK)

                wait_idx(1)
                process_super(1, base0 + (sup + 1) * SK)

            plsc.subcore_barrier()
            pltpu.sync_copy(acc_sh.at[pl.ds(row0, NROWS_W)],
                            o_h.at[pl.ds(row0, NROWS_W), pl.ds(col, Q)])
            plsc.subcore_barrier()

        for c in range(NC):
            @pl.when(cid == c)
            def _():
                one_quarter(2 * c)
                one_quarter(2 * c + 1)

    return k(m.reshape(N * NQ, Q), *src4, dst, zeros)


def _pool_sc(h, gids, zeros):
    H = D_HID // NC

    @functools.partial(
        pl.kernel,
        out_type=jax.ShapeDtypeStruct((G, D_HID), f32),
        mesh=_mesh,
        compiler_params=_sc_params,
        scratch_types=[
            pltpu.VMEM((PK,), jnp.int32),
            pltpu.VMEM((PK, H), f32),
            pltpu.VMEM_SHARED((G, H), f32),
            pltpu.SemaphoreType.DMA,
        ],
    )
    def k(h_h, g_h, z_h, o_h, ids_v, rows_v, acc_sh, sem):
        cid = lax.axis_index("c")
        sid = lax.axis_index("s")
        g0 = sid * GROWS_W
        nchunks = (NPCHUNK - sid + NS - 1) // NS

        def one_half(c):
            col = c * H
            pltpu.sync_copy(z_h.at[pl.ds(g0, GROWS_W), pl.ds(0, H)],
                            acc_sh.at[pl.ds(g0, GROWS_W)])
            plsc.subcore_barrier()

            @pl.loop(0, nchunks)
            def _(j):
                base = (sid + j * NS) * PK
                pltpu.sync_copy(g_h.at[pl.ds(base, PK)], ids_v)
                pltpu.sync_copy(h_h.at[pl.ds(base, PK), pl.ds(col, H)], rows_v)
                pltpu.sync_copy(rows_v, acc_sh.at[ids_v], add=True)

            plsc.subcore_barrier()
            pltpu.sync_copy(acc_sh.at[pl.ds(g0, GROWS_W)],
                            o_h.at[pl.ds(g0, GROWS_W), pl.ds(col, H)])
            plsc.subcore_barrier()

        for c in range(NC):
            @pl.when(cid == c)
            def _():
                one_half(c)

    return k(h, gids, zeros)


def _tc_layer1(x, W_map, W1_conv, W1_res, edge_index):
    EB = 32768

    def body(x_ref, e_ref, wm_ref, wc_ref, wr_ref, m_ref, r_ref,
             s0, s1, s2, s3, dc):
        h0 = jnp.dot(x_ref[...], wm_ref[...], preferred_element_type=f32)
        m_ref[...] = jnp.dot(h0, wc_ref[...], preferred_element_type=f32)
        r_ref[...] = jax.nn.relu(jnp.dot(h0, wr_ref[...],
                                         preferred_element_type=f32))
        e = e_ref[...]
        s4 = e[0] * NQ
        s0[...] = s4
        s1[...] = s4 + 1
        s2[...] = s4 + 2
        s3[...] = s4 + 3
        dc[...] = e[1]

    return pl.pallas_call(
        body,
        grid=(N // RB,),
        in_specs=[
            pl.BlockSpec((RB, D_IN), lambda i: (i, 0)),
            pl.BlockSpec((2, EB), lambda i: (0, i)),
            pl.BlockSpec((D_IN, D_HID), lambda i: (0, 0)),
            pl.BlockSpec((D_HID, D_HID), lambda i: (0, 0)),
            pl.BlockSpec((D_HID, D_HID), lambda i: (0, 0)),
        ],
        out_specs=[pl.BlockSpec((RB, D_HID), lambda i: (i, 0))] * 2
        + [pl.BlockSpec((EB,), lambda i: (i,))] * 5,
        out_shape=[jax.ShapeDtypeStruct((N, D_HID), f32)] * 2
        + [jax.ShapeDtypeStruct((E,), jnp.int32)] * 5,
    )(x, edge_index, W_map, W1_conv, W1_res)


def _tc_layer2(agg1, r1, W2_conv, W2_res):
    def body(a_ref, r1_ref, wc_ref, wr_ref, m_ref, r_ref):
        h = jax.nn.relu(a_ref[...]) + r1_ref[...]
        m_ref[...] = jnp.dot(h, wc_ref[...], preferred_element_type=f32)
        r_ref[...] = jax.nn.relu(jnp.dot(h, wr_ref[...],
                                         preferred_element_type=f32))

    return pl.pallas_call(
        body,
        grid=(N // RB,),
        in_specs=[
            pl.BlockSpec((RB, D_HID), lambda i: (i, 0)),
            pl.BlockSpec((RB, D_HID), lambda i: (i, 0)),
            pl.BlockSpec((D_HID, D_HID), lambda i: (0, 0)),
            pl.BlockSpec((D_HID, D_HID), lambda i: (0, 0)),
        ],
        out_specs=[pl.BlockSpec((RB, D_HID), lambda i: (i, 0))] * 2,
        out_shape=[jax.ShapeDtypeStruct((N, D_HID), f32)] * 2,
    )(agg1, r1, W2_conv, W2_res)


def _tc_h2(agg2, r2):
    def body(a_ref, r_ref, o_ref):
        o_ref[...] = jax.nn.relu(a_ref[...]) + r_ref[...]

    return pl.pallas_call(
        body,
        grid=(N // RB,),
        in_specs=[pl.BlockSpec((RB, D_HID), lambda i: (i, 0))] * 2,
        out_specs=pl.BlockSpec((RB, D_HID), lambda i: (i, 0)),
        out_shape=jax.ShapeDtypeStruct((N, D_HID), f32),
    )(agg2, r2)


def kernel(x, edge_index, node_graph_ids, W_map, W1_conv, W1_res, W2_conv,
           W2_res):
    zeros = jnp.zeros((N_PAD, D_HID // NC), f32)

    m1, r1, s0, s1, s2, s3, dst = _tc_layer1(x, W_map, W1_conv, W1_res,
                                             edge_index)
    src4 = (s0, s1, s2, s3)
    agg1 = _edge_segsum_sc(m1, src4, dst, zeros)
    m2, r2 = _tc_layer2(agg1, r1, W2_conv, W2_res)
    agg2 = _edge_segsum_sc(m2, src4, dst, zeros)
    h2 = _tc_h2(agg2, r2)
    return _pool_sc(h2, node_graph_ids, zeros)
